# baseline (speedup 1.0000x reference)
import jax
import jax.numpy as jnp
from jax import lax
from jax.experimental import pallas as pl
from jax.experimental.pallas import tpu as pltpu
from jax.experimental.pallas import tpu_sc as plsc

VOCAB = 1000000
EMBED_DIM = 64
OUTPUT_DIM = 128
BATCH = 4096
HIST = 200

NUM_CORES = 2
NUM_SUBCORES = 16
NUM_WORKERS = NUM_CORES * NUM_SUBCORES
ROWS_PER_WORKER = BATCH // NUM_WORKERS
LANES = 16
CHUNK0 = 128
CHUNK1 = HIST - CHUNK0

_sc_mesh = plsc.VectorSubcoreMesh(
    core_axis_name="c", subcore_axis_name="s"
)


def _sc_sum_kernel(idx_hbm, emb_hbm, out_hbm, idx_v, buf0, buf1, out_v, sem):
    wid = lax.axis_index("s") * NUM_CORES + lax.axis_index("c")
    base = wid * ROWS_PER_WORKER

    pltpu.sync_copy(idx_hbm.at[wid], idx_v)

    def body(b, carry):
        g0 = pltpu.async_copy(
            emb_hbm.at[idx_v.at[b, pl.ds(0, CHUNK0)]], buf0, sem
        )
        g1 = pltpu.async_copy(
            emb_hbm.at[idx_v.at[b, pl.ds(CHUNK0, CHUNK1)]], buf1, sem
        )
        g0.wait()
        g1.wait()
        for k in range(EMBED_DIM // LANES):
            sl = pl.ds(k * LANES, LANES)
            acc = buf0[0, sl]
            for j in range(1, CHUNK0):
                acc = acc + buf0[j, sl]
            for j in range(CHUNK1):
                acc = acc + buf1[j, sl]
            out_v[b, sl] = acc
        return carry

    lax.fori_loop(0, ROWS_PER_WORKER, body, 0)

    pltpu.sync_copy(out_v, out_hbm.at[pl.ds(base, ROWS_PER_WORKER), :])


_sc_sum = pl.kernel(
    _sc_sum_kernel,
    out_type=jax.ShapeDtypeStruct((BATCH, EMBED_DIM), jnp.float32),
    mesh=_sc_mesh,
    scratch_types=[
        pltpu.VMEM((ROWS_PER_WORKER, HIST), jnp.int32),
        pltpu.VMEM((CHUNK0, EMBED_DIM), jnp.float32),
        pltpu.VMEM((CHUNK1, EMBED_DIM), jnp.float32),
        pltpu.VMEM((ROWS_PER_WORKER, EMBED_DIM), jnp.float32),
        pltpu.SemaphoreType.DMA,
    ],
    compiler_params=pltpu.CompilerParams(use_tc_tiling_on_sc=False),
)


def _tc_linear_kernel(x_ref, w_ref, b_ref, o_ref):
    x = x_ref[...] * jnp.float32(1.0 / HIST)
    o_ref[...] = (
        jnp.dot(x, w_ref[...], preferred_element_type=jnp.float32)
        + b_ref[...]
    )


_TC_BLOCK = 512


def _tc_linear(x, W, b2d):
    return pl.pallas_call(
        _tc_linear_kernel,
        grid=(BATCH // _TC_BLOCK,),
        in_specs=[
            pl.BlockSpec((_TC_BLOCK, EMBED_DIM), lambda i: (i, 0)),
            pl.BlockSpec((EMBED_DIM, OUTPUT_DIM), lambda i: (0, 0)),
            pl.BlockSpec((1, OUTPUT_DIM), lambda i: (0, 0)),
        ],
        out_specs=pl.BlockSpec((_TC_BLOCK, OUTPUT_DIM), lambda i: (i, 0)),
        out_shape=jax.ShapeDtypeStruct((BATCH, OUTPUT_DIM), jnp.float32),
    )(x, W, b2d)


def kernel(word_indices, embedding, W, b):
    idx = word_indices.astype(jnp.int32).reshape(
        NUM_WORKERS, ROWS_PER_WORKER, HIST
    )
    sums = _sc_sum(idx, embedding)
    return _tc_linear(sums, W, b.reshape(1, OUTPUT_DIM))


# speedup vs baseline: 1.1188x; 1.1188x over previous
import jax
import jax.numpy as jnp
from jax import lax
from jax.experimental import pallas as pl
from jax.experimental.pallas import tpu as pltpu
from jax.experimental.pallas import tpu_sc as plsc

VOCAB = 1000000
EMBED_DIM = 64
OUTPUT_DIM = 128
BATCH = 4096
HIST = 200

NUM_CORES = 2
NUM_SUBCORES = 16
NUM_WORKERS = NUM_CORES * NUM_SUBCORES
ROWS_PER_WORKER = BATCH // NUM_WORKERS
LANES = 16
CHUNK0 = 128
CHUNK1 = HIST - CHUNK0

_sc_mesh = plsc.VectorSubcoreMesh(
    core_axis_name="c", subcore_axis_name="s"
)


def _sc_sum_kernel(
    idx_hbm, emb_hbm, out_hbm, idx_v, buf_a, buf_b, out_v, sem_a, sem_b
):
    wid = lax.axis_index("s") * NUM_CORES + lax.axis_index("c")
    base = wid * ROWS_PER_WORKER

    pltpu.sync_copy(idx_hbm.at[wid], idx_v)

    def issue(b, buf, sem):
        pltpu.async_copy(
            emb_hbm.at[idx_v.at[b, pl.ds(0, CHUNK0)]],
            buf.at[pl.ds(0, CHUNK0), :],
            sem,
        )
        pltpu.async_copy(
            emb_hbm.at[idx_v.at[b, pl.ds(CHUNK0, CHUNK1)]],
            buf.at[pl.ds(CHUNK0, CHUNK1), :],
            sem,
        )

    def drain(buf, sem):
        pltpu.make_async_copy(
            emb_hbm.at[pl.ds(0, HIST), :], buf, sem
        ).wait()

    def accumulate(b, buf):
        for k in range(EMBED_DIM // LANES):
            sl = pl.ds(k * LANES, LANES)
            acc = buf[0, sl]
            for j in range(1, HIST):
                acc = acc + buf[j, sl]
            out_v[b, sl] = acc

    issue(0, buf_a, sem_a)
    issue(1, buf_b, sem_b)

    def body(i, carry):
        b = 2 * i
        drain(buf_a, sem_a)
        accumulate(b, buf_a)

        @pl.when(b + 2 < ROWS_PER_WORKER)
        def _():
            issue(b + 2, buf_a, sem_a)

        drain(buf_b, sem_b)
        accumulate(b + 1, buf_b)

        @pl.when(b + 3 < ROWS_PER_WORKER)
        def _():
            issue(b + 3, buf_b, sem_b)

        return carry

    lax.fori_loop(0, ROWS_PER_WORKER // 2, body, 0)

    pltpu.sync_copy(out_v, out_hbm.at[pl.ds(base, ROWS_PER_WORKER), :])


_sc_sum = pl.kernel(
    _sc_sum_kernel,
    out_type=jax.ShapeDtypeStruct((BATCH, EMBED_DIM), jnp.float32),
    mesh=_sc_mesh,
    scratch_types=[
        pltpu.VMEM((ROWS_PER_WORKER, HIST), jnp.int32),
        pltpu.VMEM((HIST, EMBED_DIM), jnp.float32),
        pltpu.VMEM((HIST, EMBED_DIM), jnp.float32),
        pltpu.VMEM((ROWS_PER_WORKER, EMBED_DIM), jnp.float32),
        pltpu.SemaphoreType.DMA,
        pltpu.SemaphoreType.DMA,
    ],
    compiler_params=pltpu.CompilerParams(use_tc_tiling_on_sc=False),
)


def _tc_linear_kernel(x_ref, w_ref, b_ref, o_ref):
    x = x_ref[...] * jnp.float32(1.0 / HIST)
    o_ref[...] = (
        jnp.dot(x, w_ref[...], preferred_element_type=jnp.float32)
        + b_ref[...]
    )


_TC_BLOCK = 512


def _tc_linear(x, W, b2d):
    return pl.pallas_call(
        _tc_linear_kernel,
        grid=(BATCH // _TC_BLOCK,),
        in_specs=[
            pl.BlockSpec((_TC_BLOCK, EMBED_DIM), lambda i: (i, 0)),
            pl.BlockSpec((EMBED_DIM, OUTPUT_DIM), lambda i: (0, 0)),
            pl.BlockSpec((1, OUTPUT_DIM), lambda i: (0, 0)),
        ],
        out_specs=pl.BlockSpec((_TC_BLOCK, OUTPUT_DIM), lambda i: (i, 0)),
        out_shape=jax.ShapeDtypeStruct((BATCH, OUTPUT_DIM), jnp.float32),
    )(x, W, b2d)


def kernel(word_indices, embedding, W, b):
    idx = word_indices.astype(jnp.int32).reshape(
        NUM_WORKERS, ROWS_PER_WORKER, HIST
    )
    sums = _sc_sum(idx, embedding)
    return _tc_linear(sums, W, b.reshape(1, OUTPUT_DIM))
